# baseline (speedup 1.0000x reference)
import functools

import jax
import jax.numpy as jnp
from jax import lax
from jax.experimental import pallas as pl
from jax.experimental.pallas import tpu as pltpu
from jax.experimental.pallas import tpu_sc as plsc

_LANES = 16
_SUB_ROWS = 8
_NXBUF = 4
_NPBUF = 2


def _build_sc_add(batch, seq, dim):
    info = plsc.get_sparse_core_info()
    nc, ns = info.num_cores, info.num_subcores
    nw = nc * ns
    rows_per_worker = seq // nw
    nsub = rows_per_worker // _SUB_ROWS
    nsteps = nsub * batch
    half = dim // 2
    mesh = plsc.VectorSubcoreMesh(core_axis_name="c", subcore_axis_name="s")

    buf = lambda: pltpu.VMEM((_SUB_ROWS, dim), jnp.float32)

    @functools.partial(
        pl.kernel,
        mesh=mesh,
        out_type=jax.ShapeDtypeStruct((batch, seq, dim), jnp.float32),
        scratch_types=(
            [buf() for _ in range(_NXBUF)]
            + [buf() for _ in range(_NPBUF)]
            + [pltpu.SemaphoreType.DMA] * (2 * _NXBUF + _NPBUF)
        ),
    )
    def sc_add(x_hbm, pos_hbm, out_hbm, *refs):
        x_v = refs[:_NXBUF]
        pos_v = refs[_NXBUF : _NXBUF + _NPBUF]
        lsem = refs[_NXBUF + _NPBUF : 2 * _NXBUF + _NPBUF]
        ssem = refs[2 * _NXBUF + _NPBUF : 3 * _NXBUF + _NPBUF]
        psem = refs[3 * _NXBUF + _NPBUF :]

        wid = lax.axis_index("s") * nc + lax.axis_index("c")
        base_row = wid * rows_per_worker

        def step_row(t):
            return base_row + (t // batch) * _SUB_ROWS

        def issue_load(t):
            slot = t % _NXBUF
            return pltpu.async_copy(
                x_hbm.at[t % batch, pl.ds(step_row(t), _SUB_ROWS), :],
                x_v[slot],
                lsem[slot],
            )

        def issue_pos(sub):
            slot = sub % _NPBUF
            return pltpu.async_copy(
                pos_hbm.at[pl.ds(base_row + sub * _SUB_ROWS, _SUB_ROWS), :],
                pos_v[slot],
                psem[slot],
            )

        loads = [None] * nsteps
        stores = [None] * nsteps
        ploads = [None] * nsub

        ploads[0] = issue_pos(0)
        loads[0] = issue_load(0)
        if nsteps > 1:
            loads[1] = issue_load(1)

        for t in range(nsteps):
            slot = t % _NXBUF
            sub = t // batch
            if t % batch == 0 and sub + 1 < nsub:
                ploads[sub + 1] = issue_pos(sub + 1)
            if t + 2 < nsteps:
                if t >= 2:
                    stores[t - 2].wait()
                loads[t + 2] = issue_load(t + 2)
            if t % batch == 0:
                ploads[sub].wait()
            loads[t].wait()

            xb = x_v[slot]
            pb = pos_v[sub % _NPBUF]

            def row_body(r, carry):
                def half_body(h, carry2):
                    o0 = h * half
                    for j in range(half // _LANES):
                        o = o0 + j * _LANES
                        xb[r, pl.ds(o, _LANES)] = (
                            xb[r, pl.ds(o, _LANES)] + pb[r, pl.ds(o, _LANES)]
                        )
                    return carry2

                return lax.fori_loop(0, 2, half_body, carry)

            lax.fori_loop(0, _SUB_ROWS, row_body, 0)

            stores[t] = pltpu.async_copy(
                xb,
                out_hbm.at[t % batch, pl.ds(step_row(t), _SUB_ROWS), :],
                ssem[slot],
            )

        for t in range(max(0, nsteps - 2), nsteps):
            stores[t].wait()

    return sc_add


@jax.jit
def kernel(x, pos_table):
    b, s, d = x.shape
    return _build_sc_add(b, s, d)(x, pos_table)


# speedup vs baseline: 2.2885x; 2.2885x over previous
import functools

import jax
import jax.numpy as jnp
from jax import lax
from jax.experimental import pallas as pl
from jax.experimental.pallas import tpu as pltpu
from jax.experimental.pallas import tpu_sc as plsc

_LANES = 16
_SUB_ROWS = 8
_NXBUF = 4
_NPBUF = 2


def _build_sc_add(batch, seq, dim):
    info = plsc.get_sparse_core_info()
    nc, ns = info.num_cores, info.num_subcores
    nw = nc * ns
    rows_per_worker = seq // nw
    nsub = rows_per_worker // _SUB_ROWS
    nsteps = nsub * batch
    half = dim // 2
    mesh = plsc.VectorSubcoreMesh(core_axis_name="c", subcore_axis_name="s")

    buf = lambda: pltpu.VMEM((_SUB_ROWS, dim), jnp.float32)

    @functools.partial(
        pl.kernel,
        mesh=mesh,
        out_type=jax.ShapeDtypeStruct((batch, seq, dim), jnp.float32),
        scratch_types=(
            [buf() for _ in range(_NXBUF)]
            + [buf() for _ in range(_NPBUF)]
            + [pltpu.SemaphoreType.DMA] * (2 * _NXBUF + _NPBUF)
        ),
    )
    def sc_add(x_hbm, pos_hbm, out_hbm, *refs):
        x_v = refs[:_NXBUF]
        pos_v = refs[_NXBUF : _NXBUF + _NPBUF]
        lsem = refs[_NXBUF + _NPBUF : 2 * _NXBUF + _NPBUF]
        ssem = refs[2 * _NXBUF + _NPBUF : 3 * _NXBUF + _NPBUF]
        psem = refs[3 * _NXBUF + _NPBUF :]

        wid = lax.axis_index("s") * nc + lax.axis_index("c")
        base_row = wid * rows_per_worker

        def step_row(t):
            return base_row + (t // batch) * _SUB_ROWS

        def issue_load(t):
            slot = t % _NXBUF
            return pltpu.async_copy(
                x_hbm.at[t % batch, pl.ds(step_row(t), _SUB_ROWS), :],
                x_v[slot],
                lsem[slot],
            )

        def issue_pos(sub):
            slot = sub % _NPBUF
            return pltpu.async_copy(
                pos_hbm.at[pl.ds(base_row + sub * _SUB_ROWS, _SUB_ROWS), :],
                pos_v[slot],
                psem[slot],
            )

        loads = [None] * nsteps
        stores = [None] * nsteps
        ploads = [None] * nsub

        ploads[0] = issue_pos(0)
        loads[0] = issue_load(0)
        if nsteps > 1:
            loads[1] = issue_load(1)

        for t in range(nsteps):
            slot = t % _NXBUF
            sub = t // batch
            if t % batch == 0 and sub + 1 < nsub:
                ploads[sub + 1] = issue_pos(sub + 1)
            if t + 2 < nsteps:
                if t >= 2:
                    stores[t - 2].wait()
                loads[t + 2] = issue_load(t + 2)
            if t % batch == 0:
                ploads[sub].wait()
            loads[t].wait()

            xb = x_v[slot]
            pb = pos_v[sub % _NPBUF]

            @plsc.parallel_loop(0, _SUB_ROWS)
            def row_body(r):
                @plsc.parallel_loop(0, dim, step=_LANES, unroll=8)
                def slice_body(o):
                    xb[r, pl.ds(o, _LANES)] = (
                        xb[r, pl.ds(o, _LANES)] + pb[r, pl.ds(o, _LANES)]
                    )

            stores[t] = pltpu.async_copy(
                xb,
                out_hbm.at[t % batch, pl.ds(step_row(t), _SUB_ROWS), :],
                ssem[slot],
            )

        for t in range(max(0, nsteps - 2), nsteps):
            stores[t].wait()

    return sc_add


@jax.jit
def kernel(x, pos_table):
    b, s, d = x.shape
    return _build_sc_add(b, s, d)(x, pos_table)


# speedup vs baseline: 2.3778x; 1.0390x over previous
import functools

import jax
import jax.numpy as jnp
from jax import lax
from jax.experimental import pallas as pl
from jax.experimental.pallas import tpu as pltpu
from jax.experimental.pallas import tpu_sc as plsc

_LANES = 16
_SUB_ROWS = 16
_NXBUF = 4
_NPBUF = 2


def _build_sc_add(batch, seq, dim):
    info = plsc.get_sparse_core_info()
    nc, ns = info.num_cores, info.num_subcores
    nw = nc * ns
    rows_per_worker = seq // nw
    nsub = rows_per_worker // _SUB_ROWS
    nsteps = nsub * batch
    half = dim // 2
    mesh = plsc.VectorSubcoreMesh(core_axis_name="c", subcore_axis_name="s")

    buf = lambda: pltpu.VMEM((_SUB_ROWS, dim), jnp.float32)

    @functools.partial(
        pl.kernel,
        mesh=mesh,
        out_type=jax.ShapeDtypeStruct((batch, seq, dim), jnp.float32),
        scratch_types=(
            [buf() for _ in range(_NXBUF)]
            + [buf() for _ in range(_NPBUF)]
            + [pltpu.SemaphoreType.DMA] * (2 * _NXBUF + _NPBUF)
        ),
    )
    def sc_add(x_hbm, pos_hbm, out_hbm, *refs):
        x_v = refs[:_NXBUF]
        pos_v = refs[_NXBUF : _NXBUF + _NPBUF]
        lsem = refs[_NXBUF + _NPBUF : 2 * _NXBUF + _NPBUF]
        ssem = refs[2 * _NXBUF + _NPBUF : 3 * _NXBUF + _NPBUF]
        psem = refs[3 * _NXBUF + _NPBUF :]

        wid = lax.axis_index("s") * nc + lax.axis_index("c")
        base_row = wid * rows_per_worker

        def step_row(t):
            return base_row + (t // batch) * _SUB_ROWS

        def issue_load(t):
            slot = t % _NXBUF
            return pltpu.async_copy(
                x_hbm.at[t % batch, pl.ds(step_row(t), _SUB_ROWS), :],
                x_v[slot],
                lsem[slot],
            )

        def issue_pos(sub):
            slot = sub % _NPBUF
            return pltpu.async_copy(
                pos_hbm.at[pl.ds(base_row + sub * _SUB_ROWS, _SUB_ROWS), :],
                pos_v[slot],
                psem[slot],
            )

        loads = [None] * nsteps
        stores = [None] * nsteps
        ploads = [None] * nsub

        ploads[0] = issue_pos(0)
        loads[0] = issue_load(0)
        if nsteps > 1:
            loads[1] = issue_load(1)

        for t in range(nsteps):
            slot = t % _NXBUF
            sub = t // batch
            if t % batch == 0 and sub + 1 < nsub:
                ploads[sub + 1] = issue_pos(sub + 1)
            if t + 2 < nsteps:
                if t >= 2:
                    stores[t - 2].wait()
                loads[t + 2] = issue_load(t + 2)
            if t % batch == 0:
                ploads[sub].wait()
            loads[t].wait()

            xb = x_v[slot]
            pb = pos_v[sub % _NPBUF]

            @plsc.parallel_loop(0, _SUB_ROWS)
            def row_body(r):
                @plsc.parallel_loop(0, dim, step=_LANES, unroll=8)
                def slice_body(o):
                    xb[r, pl.ds(o, _LANES)] = (
                        xb[r, pl.ds(o, _LANES)] + pb[r, pl.ds(o, _LANES)]
                    )

            stores[t] = pltpu.async_copy(
                xb,
                out_hbm.at[t % batch, pl.ds(step_row(t), _SUB_ROWS), :],
                ssem[slot],
            )

        for t in range(max(0, nsteps - 2), nsteps):
            stores[t].wait()

    return sc_add


@jax.jit
def kernel(x, pos_table):
    b, s, d = x.shape
    return _build_sc_add(b, s, d)(x, pos_table)
